# Initial kernel scaffold; baseline (speedup 1.0000x reference)
import jax
import jax.numpy as jnp
from jax.experimental import pallas as pl


def kernel(xs, weight):
    raise NotImplementedError("write your pallas kernel here")


# speedup vs baseline: 1.4952x; 1.4952x over previous
import functools

import jax
import jax.numpy as jnp
from jax import lax
from jax.experimental import pallas as pl
from jax.experimental.pallas import tpu as pltpu
from jax.experimental.pallas import tpu_sc as plsc

D = 32
NC, NS = 2, 16
NW = NC * NS
CHUNK = 128
STREAMS = 10
ROWS = CHUNK * STREAMS


def kernel(xs, weight):
    B, H = xs.shape
    TOT = B * H
    per_w = TOT // NW
    n_chunks = per_w // CHUNK
    iters = per_w // ROWS
    assert per_w * NW == TOT and n_chunks * CHUNK == per_w and iters * ROWS == per_w

    xs_flat = xs.reshape(TOT // CHUNK, CHUNK).astype(jnp.int32)
    mesh = plsc.VectorSubcoreMesh(core_axis_name="c", subcore_axis_name="s")

    @functools.partial(
        pl.kernel,
        out_type=jax.ShapeDtypeStruct((TOT, D), jnp.float32),
        mesh=mesh,
        scratch_types=[
            pltpu.VMEM((n_chunks, CHUNK), jnp.int32),
            pltpu.VMEM((2, ROWS, D), jnp.float32),
            pltpu.SemaphoreType.DMA,
            pltpu.SemaphoreType.DMA,
            pltpu.SemaphoreType.DMA,
            pltpu.SemaphoreType.DMA,
        ],
        compiler_params=pltpu.CompilerParams(use_tc_tiling_on_sc=False),
    )
    def run(xs_hbm, w_hbm, out_hbm, idx_v, rows_v, gsem0, gsem1, osem0, osem1):
        wid = lax.axis_index("s") * NC + lax.axis_index("c")
        row0 = wid * n_chunks
        out0 = wid * per_w

        gsems = (gsem0, gsem1)
        osems = (osem0, osem1)

        pltpu.sync_copy(xs_hbm.at[pl.ds(row0, n_chunks)], idx_v)

        def fill(slot, it):
            for s in range(STREAMS):
                j = it * STREAMS + s
                pltpu.async_copy(
                    w_hbm.at[idx_v.at[j]],
                    rows_v.at[slot].at[pl.ds(s * CHUNK, CHUNK)],
                    gsems[slot],
                )

        def gwait(slot):
            pltpu.make_async_copy(
                w_hbm.at[pl.ds(0, ROWS)], rows_v.at[slot], gsems[slot]
            ).wait()

        def ostart(slot, it):
            pltpu.async_copy(
                rows_v.at[slot],
                out_hbm.at[pl.ds(out0 + it * ROWS, ROWS)],
                osems[slot],
            )

        def owait(slot):
            pltpu.make_async_copy(
                rows_v.at[slot], out_hbm.at[pl.ds(out0, ROWS)], osems[slot]
            ).wait()

        fill(0, 0)

        @pl.loop(0, iters, step=2)
        def _(g):
            gwait(0)
            ostart(0, g)

            @pl.when(g > 0)
            def _():
                owait(1)

            fill(1, g + 1)

            gwait(1)
            ostart(1, g + 1)

            @pl.when(g + 2 < iters)
            def _():
                owait(0)
                fill(0, g + 2)

        owait(0)
        owait(1)

    out = run(xs_flat, weight)
    return out.reshape(B, H, D)


# speedup vs baseline: 1.4999x; 1.0032x over previous
import functools

import jax
import jax.numpy as jnp
from jax import lax
from jax.experimental import pallas as pl
from jax.experimental.pallas import tpu as pltpu
from jax.experimental.pallas import tpu_sc as plsc

D = 32
NC, NS = 2, 16
NW = NC * NS
CHUNK = 128
STREAMS = 10
ROWS = CHUNK * STREAMS


def kernel(xs, weight):
    B, H = xs.shape
    TOT = B * H
    per_w = TOT // NW
    n_chunks = per_w // CHUNK
    iters = per_w // ROWS
    assert per_w * NW == TOT and n_chunks * CHUNK == per_w and iters * ROWS == per_w

    xs_flat = xs.reshape(TOT).astype(jnp.int32)
    mesh = plsc.VectorSubcoreMesh(core_axis_name="c", subcore_axis_name="s")

    @functools.partial(
        pl.kernel,
        out_type=jax.ShapeDtypeStruct((TOT, D), jnp.float32),
        mesh=mesh,
        scratch_types=[
            pltpu.VMEM((per_w,), jnp.int32),
            pltpu.VMEM((2, ROWS, D), jnp.float32),
            pltpu.SemaphoreType.DMA,
            pltpu.SemaphoreType.DMA,
            pltpu.SemaphoreType.DMA,
            pltpu.SemaphoreType.DMA,
        ],
        compiler_params=pltpu.CompilerParams(use_tc_tiling_on_sc=False),
    )
    def run(xs_hbm, w_hbm, out_hbm, idx_v, rows_v, gsem0, gsem1, osem0, osem1):
        wid = lax.axis_index("s") * NC + lax.axis_index("c")
        base = wid * per_w

        gsems = (gsem0, gsem1)
        osems = (osem0, osem1)

        pltpu.sync_copy(xs_hbm.at[pl.ds(base, per_w)], idx_v)

        def fill(slot, it):
            pltpu.async_copy(
                w_hbm.at[idx_v.at[pl.ds(it * ROWS, ROWS)]],
                rows_v.at[slot],
                gsems[slot],
            )

        def gwait(slot):
            pltpu.make_async_copy(
                w_hbm.at[pl.ds(0, ROWS)], rows_v.at[slot], gsems[slot]
            ).wait()

        def ostart(slot, it):
            pltpu.async_copy(
                rows_v.at[slot],
                out_hbm.at[pl.ds(base + it * ROWS, ROWS)],
                osems[slot],
            )

        def owait(slot):
            pltpu.make_async_copy(
                rows_v.at[slot], out_hbm.at[pl.ds(base, ROWS)], osems[slot]
            ).wait()

        fill(0, 0)
        fill(1, 1)

        @pl.loop(0, iters, step=2)
        def _(g):
            gwait(0)
            ostart(0, g)

            @pl.when(g + 2 < iters)
            def _():
                owait(0)
                fill(0, g + 2)

            gwait(1)
            ostart(1, g + 1)

            @pl.when(g + 3 < iters)
            def _():
                owait(1)
                fill(1, g + 3)

        owait(0)
        owait(1)

    out = run(xs_flat, weight)
    return out.reshape(B, H, D)
